# baseline (speedup 1.0000x reference)
import functools

import jax
import jax.numpy as jnp
from jax import lax
from jax.experimental import pallas as pl
from jax.experimental.pallas import tpu as pltpu
from jax.experimental.pallas import tpu_sc as plsc

B = 4096
L = 50
D = 128
H = 256

NC = 2
NS = 16
NW = NC * NS
BPW = B // NW
CB = 8
NCHUNK = BPW // CB
NLANE = 16
ND = D // NLANE


def _pool_body(nsplit, split, x_hbm, table_hbm, out_hbm, idx_all, rows_a,
               rows_b, pooled_v, sem_a, sem_b):
    bpw = BPW // nsplit
    nchunk = bpw // CB
    wid = lax.axis_index("s") * NC + lax.axis_index("c")
    obase = wid * bpw
    wbase = split * NW * bpw + obase

    pltpu.sync_copy(x_hbm.at[pl.ds(wbase * L, bpw * L)], idx_all)

    def idx_chunk(c):
        return idx_all.at[pl.ds(c * CB * L, CB * L)]

    def accum_store(c, rows_v):
        for b in range(CB):
            def acc_body(j, accs):
                row = b * L + 2 * j
                accs = tuple(
                    a + rows_v[row, pl.ds(d * NLANE, NLANE)]
                    for d, a in enumerate(accs)
                )
                return tuple(
                    a + rows_v[row + 1, pl.ds(d * NLANE, NLANE)]
                    for d, a in enumerate(accs)
                )
            accs = lax.fori_loop(
                0, L // 2, acc_body,
                tuple(jnp.zeros((NLANE,), jnp.float32) for _ in range(ND)),
            )
            for d in range(ND):
                pooled_v[b, pl.ds(d * NLANE, NLANE)] = accs[d] * (1.0 / L)
        pltpu.sync_copy(pooled_v, out_hbm.at[pl.ds(obase + c * CB, CB)])

    pltpu.async_copy(table_hbm.at[idx_chunk(0)], rows_a, sem_a)
    pltpu.async_copy(table_hbm.at[idx_chunk(1)], rows_b, sem_b)

    def g_body(g, carry):
        pltpu.make_async_copy(table_hbm.at[idx_chunk(0)], rows_a, sem_a).wait()
        accum_store(2 * g, rows_a)

        @pl.when(g < nchunk // 2 - 1)
        def _():
            pltpu.async_copy(table_hbm.at[idx_chunk(2 * g + 2)], rows_a, sem_a)

        pltpu.make_async_copy(table_hbm.at[idx_chunk(1)], rows_b, sem_b).wait()
        accum_store(2 * g + 1, rows_b)

        @pl.when(g < nchunk // 2 - 1)
        def _():
            pltpu.async_copy(table_hbm.at[idx_chunk(2 * g + 3)], rows_b, sem_b)

        return carry

    lax.fori_loop(0, nchunk // 2, g_body, 0)


def _make_pool(nsplit, split):
    bpw = BPW // nsplit
    return functools.partial(
        pl.kernel,
        out_type=jax.ShapeDtypeStruct((B // nsplit, D), jnp.float32),
        mesh=plsc.VectorSubcoreMesh(core_axis_name="c", subcore_axis_name="s"),
        scratch_types=[
            pltpu.VMEM((bpw * L,), jnp.int32),
            pltpu.VMEM((CB * L, D), jnp.float32),
            pltpu.VMEM((CB * L, D), jnp.float32),
            pltpu.VMEM((CB, D), jnp.float32),
            pltpu.SemaphoreType.DMA,
            pltpu.SemaphoreType.DMA,
        ],
    )(functools.partial(_pool_body, nsplit, split))


def _mlp_body(p_ref, w1_ref, b1_ref, w2_ref, b2_ref, *rest):
    o_ref = rest[-1]
    p = p_ref[...].astype(jnp.bfloat16)
    h = jnp.dot(p, w1_ref[...].astype(jnp.bfloat16),
                preferred_element_type=jnp.float32)
    h = jnp.maximum(h + b1_ref[...], 0.0).astype(jnp.bfloat16)
    o_ref[...] = (
        jnp.dot(h, w2_ref[...].astype(jnp.bfloat16),
                preferred_element_type=jnp.float32)
        + b2_ref[...]
    )


NSPLIT = 2
BS = B // NSPLIT

_pools = [_make_pool(NSPLIT, s) for s in range(NSPLIT)]


def _mlp_split(split, pooled, W1, b1, W2, b2, prev_out):
    args = [pooled, W1, b1, W2, b2]
    in_specs = [
        pl.BlockSpec((BS, D), lambda i: (0, 0)),
        pl.BlockSpec((D, H), lambda i: (0, 0)),
        pl.BlockSpec((1, H), lambda i: (0, 0)),
        pl.BlockSpec((H, H), lambda i: (0, 0)),
        pl.BlockSpec((1, H), lambda i: (0, 0)),
    ]
    io_aliases = {}
    if prev_out is not None:
        args.append(prev_out)
        in_specs.append(pl.BlockSpec(memory_space=pl.ANY))
        io_aliases = {5: 0}
    return pl.pallas_call(
        _mlp_body,
        grid=(1,),
        in_specs=in_specs,
        out_specs=pl.BlockSpec((BS, H), lambda i, s=split: (s, 0)),
        out_shape=jax.ShapeDtypeStruct((B, H), jnp.float32),
        input_output_aliases=io_aliases,
    )(*args)


@jax.jit
def kernel(x, emb_table, W1, b1, W2, b2):
    idx = x.reshape(-1).astype(jnp.int32)
    b1r = b1.reshape(1, H)
    b2r = b2.reshape(1, H)
    pooled = [_pools[s](idx, emb_table) for s in range(NSPLIT)]
    out = None
    for s in range(NSPLIT):
        out = _mlp_split(s, pooled[s], W1, b1r, W2, b2r, out)
    return out


# speedup vs baseline: 1.1231x; 1.1231x over previous
import functools

import jax
import jax.numpy as jnp
from jax import lax
from jax.experimental import pallas as pl
from jax.experimental.pallas import tpu as pltpu
from jax.experimental.pallas import tpu_sc as plsc

B = 4096
L = 50
D = 128
H = 256

NC = 2
NS = 16
NW = NC * NS
BPW = B // NW
CB = 8
NCHUNK = BPW // CB
NLANE = 16
ND = D // NLANE


def _pool_body(x_hbm, table_hbm, out_hbm, idx_all, rows_a, rows_b,
               pooled_v, sem_a, sem_b):
    wid = lax.axis_index("s") * NC + lax.axis_index("c")
    wbase = wid * BPW

    pltpu.sync_copy(x_hbm.at[pl.ds(wbase * L, BPW * L)], idx_all)

    def idx_chunk(c):
        return idx_all.at[pl.ds(c * CB * L, CB * L)]

    def accum_store(c, rows_v):
        for b in range(CB):
            def acc_body(j, accs):
                row = b * L + 2 * j
                accs = tuple(
                    a + rows_v[row, pl.ds(d * NLANE, NLANE)]
                    for d, a in enumerate(accs)
                )
                return tuple(
                    a + rows_v[row + 1, pl.ds(d * NLANE, NLANE)]
                    for d, a in enumerate(accs)
                )
            accs = lax.fori_loop(
                0, L // 2, acc_body,
                tuple(jnp.zeros((NLANE,), jnp.float32) for _ in range(ND)),
            )
            for d in range(ND):
                pooled_v[b, pl.ds(d * NLANE, NLANE)] = accs[d] * (1.0 / L)
        pltpu.sync_copy(pooled_v, out_hbm.at[pl.ds(wbase + c * CB, CB)])

    pltpu.async_copy(table_hbm.at[idx_chunk(0)], rows_a, sem_a)
    pltpu.async_copy(table_hbm.at[idx_chunk(1)], rows_b, sem_b)

    def g_body(g, carry):
        pltpu.make_async_copy(table_hbm.at[idx_chunk(0)], rows_a, sem_a).wait()
        accum_store(2 * g, rows_a)

        @pl.when(g < NCHUNK // 2 - 1)
        def _():
            pltpu.async_copy(table_hbm.at[idx_chunk(2 * g + 2)], rows_a, sem_a)

        pltpu.make_async_copy(table_hbm.at[idx_chunk(1)], rows_b, sem_b).wait()
        accum_store(2 * g + 1, rows_b)

        @pl.when(g < NCHUNK // 2 - 1)
        def _():
            pltpu.async_copy(table_hbm.at[idx_chunk(2 * g + 3)], rows_b, sem_b)

        return carry

    lax.fori_loop(0, NCHUNK // 2, g_body, 0)


_pool = functools.partial(
    pl.kernel,
    out_type=jax.ShapeDtypeStruct((B, D), jnp.float32),
    mesh=plsc.VectorSubcoreMesh(core_axis_name="c", subcore_axis_name="s"),
    scratch_types=[
        pltpu.VMEM((BPW * L,), jnp.int32),
        pltpu.VMEM((CB * L, D), jnp.float32),
        pltpu.VMEM((CB * L, D), jnp.float32),
        pltpu.VMEM((CB, D), jnp.float32),
        pltpu.SemaphoreType.DMA,
        pltpu.SemaphoreType.DMA,
    ],
)(_pool_body)


def _mlp_body(p_ref, w1_ref, b1_ref, w2_ref, b2_ref, o_ref):
    h = jnp.dot(p_ref[...], w1_ref[...], preferred_element_type=jnp.float32)
    h = jnp.maximum(h + b1_ref[...], 0.0)
    o_ref[...] = (
        jnp.dot(h, w2_ref[...], preferred_element_type=jnp.float32)
        + b2_ref[...]
    )


BM = 4096


def _mlp(pooled, W1, b1, W2, b2):
    return pl.pallas_call(
        _mlp_body,
        grid=(B // BM,),
        in_specs=[
            pl.BlockSpec((BM, D), lambda i: (i, 0)),
            pl.BlockSpec((D, H), lambda i: (0, 0)),
            pl.BlockSpec((1, H), lambda i: (0, 0)),
            pl.BlockSpec((H, H), lambda i: (0, 0)),
            pl.BlockSpec((1, H), lambda i: (0, 0)),
        ],
        out_specs=pl.BlockSpec((BM, H), lambda i: (i, 0)),
        out_shape=jax.ShapeDtypeStruct((B, H), jnp.float32),
    )(pooled, W1, b1, W2, b2)


@jax.jit
def kernel(x, emb_table, W1, b1, W2, b2):
    pooled = _pool(x.reshape(-1).astype(jnp.int32), emb_table)
    return _mlp(pooled, W1, b1.reshape(1, H), W2, b2.reshape(1, H))


# speedup vs baseline: 1.1344x; 1.0100x over previous
import functools

import jax
import jax.numpy as jnp
from jax import lax
from jax.experimental import pallas as pl
from jax.experimental.pallas import tpu as pltpu
from jax.experimental.pallas import tpu_sc as plsc

B = 4096
L = 50
D = 128
H = 256

NC = 2
NS = 16
NW = NC * NS
BPW = B // NW
CB = 8
NCHUNK = BPW // CB
NLANE = 16
ND = D // NLANE


def _pool_body(x_hbm, table_hbm, out_hbm, idx_all, rows_a, rows_b,
               pooled_v, sem_a, sem_b):
    wid = lax.axis_index("s") * NC + lax.axis_index("c")
    wbase = wid * BPW

    pltpu.sync_copy(x_hbm.at[pl.ds(wbase * L, BPW * L)], idx_all)

    def idx_chunk(c):
        return idx_all.at[pl.ds(c * CB * L, CB * L)]

    def accum_store(c, rows_v):
        for b in range(CB):
            def acc_body(j, accs):
                row = b * L + 2 * j
                accs = tuple(
                    a + rows_v[row, pl.ds(d * NLANE, NLANE)]
                    for d, a in enumerate(accs)
                )
                return tuple(
                    a + rows_v[row + 1, pl.ds(d * NLANE, NLANE)]
                    for d, a in enumerate(accs)
                )
            accs = lax.fori_loop(
                0, L // 2, acc_body,
                tuple(jnp.zeros((NLANE,), jnp.float32) for _ in range(ND)),
            )
            for d in range(ND):
                pooled_v[b, pl.ds(d * NLANE, NLANE)] = accs[d] * (1.0 / L)
        pltpu.sync_copy(pooled_v, out_hbm.at[pl.ds(wbase + c * CB, CB)])

    pltpu.async_copy(table_hbm.at[idx_chunk(0)], rows_a, sem_a)
    pltpu.async_copy(table_hbm.at[idx_chunk(1)], rows_b, sem_b)

    def g_body(g, carry):
        pltpu.make_async_copy(table_hbm.at[idx_chunk(0)], rows_a, sem_a).wait()
        accum_store(2 * g, rows_a)

        @pl.when(g < NCHUNK // 2 - 1)
        def _():
            pltpu.async_copy(table_hbm.at[idx_chunk(2 * g + 2)], rows_a, sem_a)

        pltpu.make_async_copy(table_hbm.at[idx_chunk(1)], rows_b, sem_b).wait()
        accum_store(2 * g + 1, rows_b)

        @pl.when(g < NCHUNK // 2 - 1)
        def _():
            pltpu.async_copy(table_hbm.at[idx_chunk(2 * g + 3)], rows_b, sem_b)

        return carry

    lax.fori_loop(0, NCHUNK // 2, g_body, 0)


_pool = functools.partial(
    pl.kernel,
    out_type=jax.ShapeDtypeStruct((B, D), jnp.float32),
    mesh=plsc.VectorSubcoreMesh(core_axis_name="c", subcore_axis_name="s"),
    scratch_types=[
        pltpu.VMEM((BPW * L,), jnp.int32),
        pltpu.VMEM((CB * L, D), jnp.float32),
        pltpu.VMEM((CB * L, D), jnp.float32),
        pltpu.VMEM((CB, D), jnp.float32),
        pltpu.SemaphoreType.DMA,
        pltpu.SemaphoreType.DMA,
    ],
)(_pool_body)


def _mlp_body(p_ref, w1_ref, b1_ref, w2_ref, b2_ref, o_ref):
    h = jnp.dot(p_ref[...], w1_ref[...], preferred_element_type=jnp.float32)
    h = jnp.maximum(h + b1_ref[...], 0.0)
    o_ref[...] = (
        jnp.dot(h, w2_ref[...], preferred_element_type=jnp.float32)
        + b2_ref[...]
    )


BM = 2048


def _mlp(pooled, W1, b1, W2, b2):
    return pl.pallas_call(
        _mlp_body,
        grid=(B // BM,),
        in_specs=[
            pl.BlockSpec((BM, D), lambda i: (i, 0)),
            pl.BlockSpec((D, H), lambda i: (0, 0)),
            pl.BlockSpec((1, H), lambda i: (0, 0)),
            pl.BlockSpec((H, H), lambda i: (0, 0)),
            pl.BlockSpec((1, H), lambda i: (0, 0)),
        ],
        out_specs=pl.BlockSpec((BM, H), lambda i: (i, 0)),
        out_shape=jax.ShapeDtypeStruct((B, H), jnp.float32),
    )(pooled, W1, b1, W2, b2)


@jax.jit
def kernel(x, emb_table, W1, b1, W2, b2):
    pooled = _pool(x.reshape(-1).astype(jnp.int32), emb_table)
    return _mlp(pooled, W1, b1.reshape(1, H), W2, b2.reshape(1, H))


# speedup vs baseline: 1.2173x; 1.0731x over previous
import functools

import jax
import jax.numpy as jnp
from jax import lax
from jax.experimental import pallas as pl
from jax.experimental.pallas import tpu as pltpu
from jax.experimental.pallas import tpu_sc as plsc

B = 4096
L = 50
D = 128
H = 256

NC = 2
NS = 16
NW = NC * NS
BPW = B // NW
CB = 4
NCHUNK = BPW // CB
NBUF = 4
NLANE = 16
ND = D // NLANE


def _pool_body(x_hbm, table_hbm, out_hbm, idx_all, r0, r1, r2, r3,
               pooled_v, s0, s1, s2, s3):
    rows = (r0, r1, r2, r3)
    sems = (s0, s1, s2, s3)
    wid = lax.axis_index("s") * NC + lax.axis_index("c")
    wbase = wid * BPW

    pltpu.sync_copy(x_hbm.at[pl.ds(wbase * L, BPW * L)], idx_all)

    def idx_chunk(c):
        return idx_all.at[pl.ds(c * CB * L, CB * L)]

    def accum_store(c, rows_v):
        for b in range(CB):
            def acc_body(j, accs):
                row = b * L + 2 * j
                accs = tuple(
                    a + rows_v[row, pl.ds(d * NLANE, NLANE)]
                    for d, a in enumerate(accs)
                )
                return tuple(
                    a + rows_v[row + 1, pl.ds(d * NLANE, NLANE)]
                    for d, a in enumerate(accs)
                )
            accs = lax.fori_loop(
                0, L // 2, acc_body,
                tuple(jnp.zeros((NLANE,), jnp.float32) for _ in range(ND)),
            )
            for d in range(ND):
                pooled_v[b, pl.ds(d * NLANE, NLANE)] = accs[d] * (1.0 / L)
        pltpu.sync_copy(pooled_v, out_hbm.at[pl.ds(wbase + c * CB, CB)])

    for k in range(NBUF):
        pltpu.async_copy(table_hbm.at[idx_chunk(k)], rows[k], sems[k])

    def g_body(g, carry):
        for k in range(NBUF):
            c = NBUF * g + k
            pltpu.make_async_copy(
                table_hbm.at[idx_chunk(0)], rows[k], sems[k]).wait()
            accum_store(c, rows[k])

            @pl.when(g < NCHUNK // NBUF - 1)
            def _():
                pltpu.async_copy(
                    table_hbm.at[idx_chunk(c + NBUF)], rows[k], sems[k])

        return carry

    lax.fori_loop(0, NCHUNK // NBUF, g_body, 0)


_pool = functools.partial(
    pl.kernel,
    out_type=jax.ShapeDtypeStruct((B, D), jnp.float32),
    mesh=plsc.VectorSubcoreMesh(core_axis_name="c", subcore_axis_name="s"),
    scratch_types=[
        pltpu.VMEM((BPW * L,), jnp.int32),
        pltpu.VMEM((CB * L, D), jnp.float32),
        pltpu.VMEM((CB * L, D), jnp.float32),
        pltpu.VMEM((CB * L, D), jnp.float32),
        pltpu.VMEM((CB * L, D), jnp.float32),
        pltpu.VMEM((CB, D), jnp.float32),
        pltpu.SemaphoreType.DMA,
        pltpu.SemaphoreType.DMA,
        pltpu.SemaphoreType.DMA,
        pltpu.SemaphoreType.DMA,
    ],
)(_pool_body)


def _mlp_body(p_ref, w1_ref, b1_ref, w2_ref, b2_ref, o_ref):
    h = jnp.dot(p_ref[...], w1_ref[...], preferred_element_type=jnp.float32)
    h = jnp.maximum(h + b1_ref[...], 0.0)
    o_ref[...] = (
        jnp.dot(h, w2_ref[...], preferred_element_type=jnp.float32)
        + b2_ref[...]
    )


BM = 2048


def _mlp(pooled, W1, b1, W2, b2):
    return pl.pallas_call(
        _mlp_body,
        grid=(B // BM,),
        in_specs=[
            pl.BlockSpec((BM, D), lambda i: (i, 0)),
            pl.BlockSpec((D, H), lambda i: (0, 0)),
            pl.BlockSpec((1, H), lambda i: (0, 0)),
            pl.BlockSpec((H, H), lambda i: (0, 0)),
            pl.BlockSpec((1, H), lambda i: (0, 0)),
        ],
        out_specs=pl.BlockSpec((BM, H), lambda i: (i, 0)),
        out_shape=jax.ShapeDtypeStruct((B, H), jnp.float32),
    )(pooled, W1, b1, W2, b2)


@jax.jit
def kernel(x, emb_table, W1, b1, W2, b2):
    pooled = _pool(x.reshape(-1).astype(jnp.int32), emb_table)
    return _mlp(pooled, W1, b1.reshape(1, H), W2, b2.reshape(1, H))


# speedup vs baseline: 1.2417x; 1.0200x over previous
import functools

import jax
import jax.numpy as jnp
from jax import lax
from jax.experimental import pallas as pl
from jax.experimental.pallas import tpu as pltpu
from jax.experimental.pallas import tpu_sc as plsc

B = 4096
L = 50
D = 128
H = 256

NC = 2
NS = 16
NW = NC * NS
BPW = B // NW
CB = 4
NCHUNK = BPW // CB
NBUF = 4
NLANE = 16
ND = D // NLANE


SPLIT = 104


def _pool_body(x_hbm, table_hbm, out_hbm, idx_all, r0, r1, r2, r3,
               pooled_v, s0, s1, s2, s3, t0, t1, t2, t3):
    rows = (r0, r1, r2, r3)
    sems = (s0, s1, s2, s3)
    sems2 = (t0, t1, t2, t3)
    wid = lax.axis_index("s") * NC + lax.axis_index("c")
    wbase = wid * BPW

    pltpu.sync_copy(x_hbm.at[pl.ds(wbase * L, BPW * L)], idx_all)

    def idx_chunk(c):
        return idx_all.at[pl.ds(c * CB * L, CB * L)]

    def accum_store(c, rows_v):
        for b in range(CB):
            def acc_body(j, accs):
                row = b * L + 2 * j
                accs = tuple(
                    a + rows_v[row, pl.ds(d * NLANE, NLANE)]
                    for d, a in enumerate(accs)
                )
                return tuple(
                    a + rows_v[row + 1, pl.ds(d * NLANE, NLANE)]
                    for d, a in enumerate(accs)
                )
            accs = lax.fori_loop(
                0, L // 2, acc_body,
                tuple(jnp.zeros((NLANE,), jnp.float32) for _ in range(ND)),
            )
            for d in range(ND):
                pooled_v[b, pl.ds(d * NLANE, NLANE)] = accs[d] * (1.0 / L)
        pltpu.sync_copy(pooled_v, out_hbm.at[pl.ds(wbase + c * CB, CB)])

    REST = CB * L - SPLIT

    def start_gather(c, k):
        base = c * CB * L
        pltpu.async_copy(
            table_hbm.at[idx_all.at[pl.ds(base, SPLIT)]],
            rows[k].at[pl.ds(0, SPLIT)], sems[k])
        pltpu.async_copy(
            table_hbm.at[idx_all.at[pl.ds(base + SPLIT, REST)]],
            rows[k].at[pl.ds(SPLIT, REST)], sems2[k])

    def wait_gather(k):
        pltpu.make_async_copy(
            table_hbm.at[idx_all.at[pl.ds(0, SPLIT)]],
            rows[k].at[pl.ds(0, SPLIT)], sems[k]).wait()
        pltpu.make_async_copy(
            table_hbm.at[idx_all.at[pl.ds(0, REST)]],
            rows[k].at[pl.ds(SPLIT, REST)], sems2[k]).wait()

    for k in range(NBUF):
        start_gather(k, k)

    def g_body(g, carry):
        for k in range(NBUF):
            c = NBUF * g + k
            wait_gather(k)
            accum_store(c, rows[k])

            @pl.when(g < NCHUNK // NBUF - 1)
            def _():
                start_gather(c + NBUF, k)

        return carry

    lax.fori_loop(0, NCHUNK // NBUF, g_body, 0)


_pool = functools.partial(
    pl.kernel,
    out_type=jax.ShapeDtypeStruct((B, D), jnp.float32),
    mesh=plsc.VectorSubcoreMesh(core_axis_name="c", subcore_axis_name="s"),
    scratch_types=[
        pltpu.VMEM((BPW * L,), jnp.int32),
        pltpu.VMEM((CB * L, D), jnp.float32),
        pltpu.VMEM((CB * L, D), jnp.float32),
        pltpu.VMEM((CB * L, D), jnp.float32),
        pltpu.VMEM((CB * L, D), jnp.float32),
        pltpu.VMEM((CB, D), jnp.float32),
        pltpu.SemaphoreType.DMA,
        pltpu.SemaphoreType.DMA,
        pltpu.SemaphoreType.DMA,
        pltpu.SemaphoreType.DMA,
        pltpu.SemaphoreType.DMA,
        pltpu.SemaphoreType.DMA,
        pltpu.SemaphoreType.DMA,
        pltpu.SemaphoreType.DMA,
    ],
)(_pool_body)


def _mlp_body(p_ref, w1_ref, b1_ref, w2_ref, b2_ref, o_ref):
    h = jnp.dot(p_ref[...], w1_ref[...], preferred_element_type=jnp.float32)
    h = jnp.maximum(h + b1_ref[...], 0.0)
    o_ref[...] = (
        jnp.dot(h, w2_ref[...], preferred_element_type=jnp.float32)
        + b2_ref[...]
    )


BM = 2048


def _mlp(pooled, W1, b1, W2, b2):
    return pl.pallas_call(
        _mlp_body,
        grid=(B // BM,),
        in_specs=[
            pl.BlockSpec((BM, D), lambda i: (i, 0)),
            pl.BlockSpec((D, H), lambda i: (0, 0)),
            pl.BlockSpec((1, H), lambda i: (0, 0)),
            pl.BlockSpec((H, H), lambda i: (0, 0)),
            pl.BlockSpec((1, H), lambda i: (0, 0)),
        ],
        out_specs=pl.BlockSpec((BM, H), lambda i: (i, 0)),
        out_shape=jax.ShapeDtypeStruct((B, H), jnp.float32),
    )(pooled, W1, b1, W2, b2)


@jax.jit
def kernel(x, emb_table, W1, b1, W2, b2):
    pooled = _pool(x.reshape(-1).astype(jnp.int32), emb_table)
    return _mlp(pooled, W1, b1.reshape(1, H), W2, b2.reshape(1, H))
